# Initial kernel scaffold; baseline (speedup 1.0000x reference)
import jax
import jax.numpy as jnp
from jax.experimental import pallas as pl


def kernel(x, pos, edge_index, W1, b1, W2, b2):
    raise NotImplementedError("write your pallas kernel here")


# speedup vs baseline: 2.8868x; 2.8868x over previous
import functools

import jax
import jax.numpy as jnp
from jax import lax
from jax.experimental import pallas as pl
from jax.experimental.pallas import tpu as pltpu
from jax.experimental.pallas import tpu_sc as plsc

NW = 32
NC = 2
B = 400
CH = 80
NCH = B // CH
LANES = 16
BE = 4000


def _sc_body(per_w, x_hbm, posx_hbm, posy_hbm, row_hbm, col_hbm,
             out_hbm, prx_hbm, pry_hbm, pcx_hbm, pcy_hbm,
             rowv, colv, xbuf, posx_v, posy_v, prxb, pryb, pcxb, pcyb,
             sem_x):
    wid = lax.axis_index("s") * NC + lax.axis_index("c")
    pltpu.sync_copy(posx_hbm, posx_v)
    pltpu.sync_copy(posy_hbm, posy_v)
    nblocks = per_w // B

    def block(b, carry):
        pltpu.sync_copy(row_hbm.at[wid, b], rowv)
        pltpu.sync_copy(col_hbm.at[wid, b], colv)
        descs = []
        for j in range(NCH):
            descs.append(pltpu.async_copy(
                x_hbm.at[colv.at[j]], xbuf.at[pl.ds(j * CH, CH)], sem_x))
        for j in range(NCH):
            for t in range(CH // LANES):
                off = j * CH + LANES * t
                ridx = rowv[j, pl.ds(LANES * t, LANES)]
                cidx = colv[j, pl.ds(LANES * t, LANES)]
                prxb[pl.ds(off, LANES)] = plsc.load_gather(posx_v, [ridx])
                pryb[pl.ds(off, LANES)] = plsc.load_gather(posy_v, [ridx])
                pcxb[pl.ds(off, LANES)] = plsc.load_gather(posx_v, [cidx])
                pcyb[pl.ds(off, LANES)] = plsc.load_gather(posy_v, [cidx])
        for d in descs:
            d.wait()
        ebase = wid * per_w + b * B
        pltpu.sync_copy(xbuf, out_hbm.at[pl.ds(ebase, B), pl.ds(0, 128)])
        pltpu.sync_copy(prxb, prx_hbm.at[pl.ds(ebase, B)])
        pltpu.sync_copy(pryb, pry_hbm.at[pl.ds(ebase, B)])
        pltpu.sync_copy(pcxb, pcx_hbm.at[pl.ds(ebase, B)])
        pltpu.sync_copy(pcyb, pcy_hbm.at[pl.ds(ebase, B)])
        return carry

    lax.fori_loop(0, nblocks, block, 0)


def _tc_body(prx_ref, pry_ref, pcx_ref, pcy_ref, w1t_ref, b1_ref, w2t_ref,
             b2_ref, obuf_ref, out_ref):
    prx = prx_ref[...]
    pry = pry_ref[...]
    pcx = pcx_ref[...]
    pcy = pcy_ref[...]
    dx = pcx - prx
    dy = pcy - pry
    dist = jnp.sqrt(dx * dx + dy * dy)
    w = w1t_ref[...]
    pre = (prx * w[0:1, :] + pry * w[1:2, :]
           + pcx * w[2:3, :] + pcy * w[3:4, :]
           + dx * w[4:5, :] + dy * w[5:6, :]
           + dist * w[6:7, :]) + b1_ref[...]
    h = jnp.maximum(pre, 0.0)
    out_ref[...] = (jnp.dot(h, w2t_ref[...],
                            preferred_element_type=jnp.float32)
                    + b2_ref[...])


def kernel(x, pos, edge_index, W1, b1, W2, b2):
    n_nodes, in_f = x.shape
    e = edge_index.shape[1]
    k = e // n_nodes
    out_f = W2.shape[0]
    width = in_f + out_f
    per_w = e // NW

    row = edge_index[0].astype(jnp.int32)
    col = edge_index[1].astype(jnp.int32)
    row2 = row.reshape(NW, per_w // B, NCH, CH)
    col2 = col.reshape(NW, per_w // B, NCH, CH)
    posx = pos[:, 0].astype(jnp.float32)
    posy = pos[:, 1].astype(jnp.float32)
    w1tp = jnp.pad(W1.T.astype(jnp.float32), ((0, 1), (0, 0)))
    b1r = b1.reshape(1, -1).astype(jnp.float32)
    w2t = W2.T.astype(jnp.float32)
    b2r = b2.reshape(1, -1).astype(jnp.float32)

    mesh = plsc.VectorSubcoreMesh(core_axis_name="c", subcore_axis_name="s")
    sc_call = pl.kernel(
        functools.partial(_sc_body, per_w),
        out_type=[
            jax.ShapeDtypeStruct((e, width), jnp.float32),
            jax.ShapeDtypeStruct((e,), jnp.float32),
            jax.ShapeDtypeStruct((e,), jnp.float32),
            jax.ShapeDtypeStruct((e,), jnp.float32),
            jax.ShapeDtypeStruct((e,), jnp.float32),
        ],
        mesh=mesh,
        compiler_params=pltpu.CompilerParams(needs_layout_passes=False),
        scratch_types=[
            pltpu.VMEM((NCH, CH), jnp.int32),
            pltpu.VMEM((NCH, CH), jnp.int32),
            pltpu.VMEM((B, in_f), jnp.float32),
            pltpu.VMEM((n_nodes,), jnp.float32),
            pltpu.VMEM((n_nodes,), jnp.float32),
            pltpu.VMEM((B,), jnp.float32),
            pltpu.VMEM((B,), jnp.float32),
            pltpu.VMEM((B,), jnp.float32),
            pltpu.VMEM((B,), jnp.float32),
            pltpu.SemaphoreType.DMA,
        ],
    )
    outbuf, prx1, pry1, pcx1, pcy1 = sc_call(x, posx, posy, row2, col2)

    out = pl.pallas_call(
        _tc_body,
        grid=(e // BE,),
        in_specs=[
            pl.BlockSpec((BE, 1), lambda i: (i, 0)),
            pl.BlockSpec((BE, 1), lambda i: (i, 0)),
            pl.BlockSpec((BE, 1), lambda i: (i, 0)),
            pl.BlockSpec((BE, 1), lambda i: (i, 0)),
            pl.BlockSpec((8, 64), lambda i: (0, 0)),
            pl.BlockSpec((1, 64), lambda i: (0, 0)),
            pl.BlockSpec((64, 128), lambda i: (0, 0)),
            pl.BlockSpec((1, 128), lambda i: (0, 0)),
            pl.BlockSpec(memory_space=pltpu.MemorySpace.HBM),
        ],
        out_specs=pl.BlockSpec((BE, in_f), lambda i: (i, 1)),
        out_shape=jax.ShapeDtypeStruct((e, width), jnp.float32),
        input_output_aliases={8: 0},
    )(prx1.reshape(e, 1), pry1.reshape(e, 1), pcx1.reshape(e, 1),
      pcy1.reshape(e, 1), w1tp, b1r, w2t, b2r, outbuf)

    return out.reshape(n_nodes, k, width)


# speedup vs baseline: 4.8200x; 1.6696x over previous
import functools

import jax
import jax.numpy as jnp
from jax import lax
from jax.experimental import pallas as pl
from jax.experimental.pallas import tpu as pltpu
from jax.experimental.pallas import tpu_sc as plsc

NW = 32
NC = 2
B = 400
CH = 80
NCH = B // CH
LANES = 16
BE = 4000


def _sc_body(per_w, x_hbm, posx_hbm, posy_hbm, row_hbm, col_hbm,
             out_hbm, g_hbm,
             rowv, colv, xbuf, gbuf, posx_v, posy_v,
             sem_x, sem_wx, sem_wg):
    wid = lax.axis_index("s") * NC + lax.axis_index("c")
    pltpu.sync_copy(posx_hbm, posx_v)
    pltpu.sync_copy(posy_hbm, posy_v)
    nblocks = per_w // B
    iota = lax.iota(jnp.int32, LANES)
    csplat = [jnp.full((LANES,), c, jnp.int32) for c in range(4)]

    def block(b, carry):
        pltpu.sync_copy(row_hbm.at[wid, b], rowv)
        pltpu.sync_copy(col_hbm.at[wid, b], colv)

        @pl.when(b > 0)
        def _():
            pltpu.make_async_copy(x_hbm.at[pl.ds(0, B)], xbuf, sem_wx).wait()

        for j in range(NCH):
            pltpu.async_copy(
                x_hbm.at[colv.at[j]], xbuf.at[pl.ds(j * CH, CH)], sem_x)

        @pl.when(b > 0)
        def _():
            pltpu.make_async_copy(g_hbm.at[pl.ds(0, B)], gbuf, sem_wg).wait()

        for j in range(NCH):
            for t in range(CH // LANES):
                off = j * CH + LANES * t
                evec = iota + off
                ridx = rowv[j, pl.ds(LANES * t, LANES)]
                cidx = colv[j, pl.ds(LANES * t, LANES)]
                plsc.store_scatter(gbuf, [evec, csplat[0]],
                                   plsc.load_gather(posx_v, [ridx]))
                plsc.store_scatter(gbuf, [evec, csplat[1]],
                                   plsc.load_gather(posy_v, [ridx]))
                plsc.store_scatter(gbuf, [evec, csplat[2]],
                                   plsc.load_gather(posx_v, [cidx]))
                plsc.store_scatter(gbuf, [evec, csplat[3]],
                                   plsc.load_gather(posy_v, [cidx]))
        pltpu.make_async_copy(x_hbm.at[pl.ds(0, B)], xbuf, sem_x).wait()
        ebase = wid * per_w + b * B
        pltpu.async_copy(xbuf, out_hbm.at[pl.ds(ebase, B), pl.ds(0, 128)],
                         sem_wx)
        pltpu.async_copy(gbuf, g_hbm.at[pl.ds(ebase, B)], sem_wg)
        return carry

    lax.fori_loop(0, nblocks, block, 0)
    pltpu.make_async_copy(x_hbm.at[pl.ds(0, B)], xbuf, sem_wx).wait()
    pltpu.make_async_copy(g_hbm.at[pl.ds(0, B)], gbuf, sem_wg).wait()


def _tc_body(g_ref, w1t_ref, b1_ref, w2t_ref, b2_ref, obuf_ref, out_ref):
    g = g_ref[...]
    prx = g[:, 0:1]
    pry = g[:, 1:2]
    pcx = g[:, 2:3]
    pcy = g[:, 3:4]
    dx = pcx - prx
    dy = pcy - pry
    dist = jnp.sqrt(dx * dx + dy * dy)
    w = w1t_ref[...]
    pre = (prx * w[0:1, :] + pry * w[1:2, :]
           + pcx * w[2:3, :] + pcy * w[3:4, :]
           + dx * w[4:5, :] + dy * w[5:6, :]
           + dist * w[6:7, :]) + b1_ref[...]
    h = jnp.maximum(pre, 0.0)
    out_ref[...] = (jnp.dot(h, w2t_ref[...],
                            preferred_element_type=jnp.float32)
                    + b2_ref[...])


def kernel(x, pos, edge_index, W1, b1, W2, b2):
    n_nodes, in_f = x.shape
    e = edge_index.shape[1]
    k = e // n_nodes
    out_f = W2.shape[0]
    width = in_f + out_f
    per_w = e // NW

    row = edge_index[0].astype(jnp.int32)
    col = edge_index[1].astype(jnp.int32)
    row2 = row.reshape(NW, per_w // B, NCH, CH)
    col2 = col.reshape(NW, per_w // B, NCH, CH)
    posx = pos[:, 0].astype(jnp.float32)
    posy = pos[:, 1].astype(jnp.float32)
    w1tp = jnp.pad(W1.T.astype(jnp.float32), ((0, 1), (0, 0)))
    b1r = b1.reshape(1, -1).astype(jnp.float32)
    w2t = W2.T.astype(jnp.float32)
    b2r = b2.reshape(1, -1).astype(jnp.float32)

    mesh = plsc.VectorSubcoreMesh(core_axis_name="c", subcore_axis_name="s")
    sc_call = pl.kernel(
        functools.partial(_sc_body, per_w),
        out_type=[
            jax.ShapeDtypeStruct((e, width), jnp.float32),
            jax.ShapeDtypeStruct((e, 128), jnp.float32),
        ],
        mesh=mesh,
        compiler_params=pltpu.CompilerParams(needs_layout_passes=False),
        scratch_types=[
            pltpu.VMEM((NCH, CH), jnp.int32),
            pltpu.VMEM((NCH, CH), jnp.int32),
            pltpu.VMEM((B, in_f), jnp.float32),
            pltpu.VMEM((B, 128), jnp.float32),
            pltpu.VMEM((n_nodes,), jnp.float32),
            pltpu.VMEM((n_nodes,), jnp.float32),
            pltpu.SemaphoreType.DMA,
            pltpu.SemaphoreType.DMA,
            pltpu.SemaphoreType.DMA,
        ],
    )
    outbuf, gfeat = sc_call(x, posx, posy, row2, col2)

    out = pl.pallas_call(
        _tc_body,
        grid=(e // BE,),
        in_specs=[
            pl.BlockSpec((BE, 128), lambda i: (i, 0)),
            pl.BlockSpec((8, 64), lambda i: (0, 0)),
            pl.BlockSpec((1, 64), lambda i: (0, 0)),
            pl.BlockSpec((64, 128), lambda i: (0, 0)),
            pl.BlockSpec((1, 128), lambda i: (0, 0)),
            pl.BlockSpec(memory_space=pltpu.MemorySpace.HBM),
        ],
        out_specs=pl.BlockSpec((BE, in_f), lambda i: (i, 1)),
        out_shape=jax.ShapeDtypeStruct((e, width), jnp.float32),
        input_output_aliases={5: 0},
    )(gfeat, w1tp, b1r, w2t, b2r, outbuf)

    return out.reshape(n_nodes, k, width)
